# Initial kernel scaffold; baseline (speedup 1.0000x reference)
import jax
import jax.numpy as jnp
from jax.experimental import pallas as pl


def kernel(flat, cu_seqlens, W1, b1, W2, b2, W3, b3, V1, c1, V2, c2, V3, c3):
    raise NotImplementedError("write your pallas kernel here")


# speedup vs baseline: 4.8298x; 4.8298x over previous
import jax
import jax.numpy as jnp
from jax.experimental import pallas as pl
from jax.experimental.pallas import tpu as pltpu

_TOT = 32768
_D = 3
_NSEG = 16
_BS = 2048
_GRID = _TOT // _BS


def _fused(cu_ref, flat_ref, w1, b1, w2, b2, w3, b3,
           v1, c1, v2, c2, v3, c3, inv_ref, out_ref, acc_ref):
    i = pl.program_id(0)

    @pl.when(i == 0)
    def _init():
        acc_ref[...] = jnp.zeros_like(acc_ref)

    x = flat_ref[...]
    h = jnp.maximum(jnp.dot(x, w1[...], preferred_element_type=jnp.float32)
                    + b1[...], 0.0)
    h = jnp.maximum(jnp.dot(h, w2[...], preferred_element_type=jnp.float32)
                    + b2[...], 0.0)
    h = jnp.maximum(jnp.dot(h, w3[...], preferred_element_type=jnp.float32)
                    + b3[...], 0.0)

    rows = i * _BS + jax.lax.broadcasted_iota(jnp.int32, (_BS, 1), 0)
    seg = jnp.zeros((_BS, 1), jnp.int32)
    for j in range(1, _NSEG):
        seg = seg + (rows >= cu_ref[j]).astype(jnp.int32)
    onehot = (seg == jax.lax.broadcasted_iota(jnp.int32, (_BS, _NSEG), 1)
              ).astype(jnp.float32)
    part = jax.lax.dot_general(onehot, h, (((0,), (0,)), ((), ())),
                               preferred_element_type=jnp.float32)
    acc_ref[...] += part

    @pl.when(i == _GRID - 1)
    def _finish():
        pooled = acc_ref[...] * inv_ref[...]
        o = jnp.maximum(jnp.dot(pooled, v1[...],
                                preferred_element_type=jnp.float32) + c1[...], 0.0)
        o = jnp.maximum(jnp.dot(o, v2[...],
                                preferred_element_type=jnp.float32) + c2[...], 0.0)
        out_ref[...] = jnp.dot(o, v3[...],
                               preferred_element_type=jnp.float32) + c3[...]


def kernel(flat, cu_seqlens, W1, b1, W2, b2, W3, b3, V1, c1, V2, c2, V3, c3):
    counts = (cu_seqlens[1:] - cu_seqlens[:-1]).astype(jnp.float32)
    inv = (1.0 / jnp.maximum(counts, 1.0)).reshape(_NSEG, 1)

    full = lambda a: pl.BlockSpec(a.shape, lambda i: (0,) * a.ndim)
    b1r, b2r, b3r = b1.reshape(1, -1), b2.reshape(1, -1), b3.reshape(1, -1)
    c1r, c2r, c3r = c1.reshape(1, -1), c2.reshape(1, -1), c3.reshape(1, -1)

    return pl.pallas_call(
        _fused,
        grid=(_GRID,),
        in_specs=[
            pl.BlockSpec(memory_space=pltpu.SMEM),
            pl.BlockSpec((_BS, _D), lambda i: (i, 0)),
            full(W1), full(b1r), full(W2), full(b2r), full(W3), full(b3r),
            full(V1), full(c1r), full(V2), full(c2r), full(V3), full(c3r),
            full(inv),
        ],
        out_specs=pl.BlockSpec((_NSEG, 25), lambda i: (0, 0)),
        out_shape=jax.ShapeDtypeStruct((_NSEG, 25), jnp.float32),
        scratch_shapes=[pltpu.VMEM((_NSEG, 128), jnp.float32)],
        compiler_params=pltpu.CompilerParams(
            dimension_semantics=("arbitrary",)),
    )(cu_seqlens, flat, W1, b1r, W2, b2r, W3, b3r,
      V1, c1r, V2, c2r, V3, c3r, inv)


# speedup vs baseline: 7.8496x; 1.6252x over previous
import jax
import jax.numpy as jnp
from jax.experimental import pallas as pl
from jax.experimental.pallas import tpu as pltpu

_TOT = 32768
_D = 3
_NSEG = 16
_BS = 2048
_GRID = _TOT // _BS


def _fused(cu_ref, flat_ref, w1, b1, w2, b2, w3, b3,
           v1, c1, v2, c2, v3, c3, inv_ref, out_ref, acc_ref):
    i = pl.program_id(0)

    @pl.when(i == 0)
    def _init():
        acc_ref[...] = jnp.zeros_like(acc_ref)

    x = flat_ref[...]
    h = jnp.maximum(jnp.dot(x, w1[...], preferred_element_type=jnp.float32)
                    + b1[...], 0.0)
    h = jnp.maximum(jnp.dot(h, w2[...], preferred_element_type=jnp.float32)
                    + b2[...], 0.0)
    h = jnp.maximum(jnp.dot(h, w3[...], preferred_element_type=jnp.float32)
                    + b3[...], 0.0)

    rows = i * _BS + jax.lax.broadcasted_iota(jnp.int32, (1, _BS), 1)
    seg = jnp.zeros((1, _BS), jnp.int32)
    for j in range(1, _NSEG):
        seg = seg + (rows >= cu_ref[j]).astype(jnp.int32)
    onehot_t = (seg == jax.lax.broadcasted_iota(jnp.int32, (_NSEG, _BS), 0)
                ).astype(jnp.float32)
    acc_ref[...] += jnp.dot(onehot_t, h, preferred_element_type=jnp.float32)

    @pl.when(i == _GRID - 1)
    def _finish():
        pooled = acc_ref[...] * inv_ref[...]
        o = jnp.maximum(jnp.dot(pooled, v1[...],
                                preferred_element_type=jnp.float32) + c1[...], 0.0)
        o = jnp.maximum(jnp.dot(o, v2[...],
                                preferred_element_type=jnp.float32) + c2[...], 0.0)
        out_ref[...] = jnp.dot(o, v3[...],
                               preferred_element_type=jnp.float32) + c3[...]


def kernel(flat, cu_seqlens, W1, b1, W2, b2, W3, b3, V1, c1, V2, c2, V3, c3):
    counts = (cu_seqlens[1:] - cu_seqlens[:-1]).astype(jnp.float32)
    inv = (1.0 / jnp.maximum(counts, 1.0)).reshape(_NSEG, 1)

    full = lambda a: pl.BlockSpec(a.shape, lambda i: (0,) * a.ndim)
    b1r, b2r, b3r = b1.reshape(1, -1), b2.reshape(1, -1), b3.reshape(1, -1)
    c1r, c2r, c3r = c1.reshape(1, -1), c2.reshape(1, -1), c3.reshape(1, -1)

    return pl.pallas_call(
        _fused,
        grid=(_GRID,),
        in_specs=[
            pl.BlockSpec(memory_space=pltpu.SMEM),
            pl.BlockSpec((_BS, _D), lambda i: (i, 0)),
            full(W1), full(b1r), full(W2), full(b2r), full(W3), full(b3r),
            full(V1), full(c1r), full(V2), full(c2r), full(V3), full(c3r),
            full(inv),
        ],
        out_specs=pl.BlockSpec((_NSEG, 25), lambda i: (0, 0)),
        out_shape=jax.ShapeDtypeStruct((_NSEG, 25), jnp.float32),
        scratch_shapes=[pltpu.VMEM((_NSEG, 128), jnp.float32)],
        compiler_params=pltpu.CompilerParams(
            dimension_semantics=("arbitrary",)),
    )(cu_seqlens, flat, W1, b1r, W2, b2r, W3, b3r,
      V1, c1r, V2, c2r, V3, c3r, inv)


# speedup vs baseline: 9.1221x; 1.1621x over previous
import jax
import jax.numpy as jnp
from jax.experimental import pallas as pl
from jax.experimental.pallas import tpu as pltpu

_TOT = 32768
_D = 3
_NSEG = 16
_BS = 4096
_GRID = _TOT // _BS


def _fused(cu_ref, flat_ref, w1, b1, w2, b2, w3, b3,
           v1, c1, v2, c2, v3, c3, inv_ref, out_ref, acc_ref):
    i = pl.program_id(0)

    @pl.when(i == 0)
    def _init():
        acc_ref[...] = jnp.zeros_like(acc_ref)

    x = flat_ref[...]
    h = jnp.maximum(jnp.dot(x, w1[...], preferred_element_type=jnp.float32)
                    + b1[...], 0.0)
    h = jnp.maximum(jnp.dot(h, w2[...], preferred_element_type=jnp.float32)
                    + b2[...], 0.0)
    h = jnp.maximum(jnp.dot(h, w3[...], preferred_element_type=jnp.float32)
                    + b3[...], 0.0)

    rows = i * _BS + jax.lax.broadcasted_iota(jnp.int32, (1, _BS), 1)
    seg = jnp.zeros((1, _BS), jnp.int32)
    for j in range(1, _NSEG):
        seg = seg + (rows >= cu_ref[j]).astype(jnp.int32)
    onehot_t = (seg == jax.lax.broadcasted_iota(jnp.int32, (_NSEG, _BS), 0)
                ).astype(jnp.float32)
    acc_ref[...] += jnp.dot(onehot_t, h, preferred_element_type=jnp.float32)

    @pl.when(i == _GRID - 1)
    def _finish():
        pooled = acc_ref[...] * inv_ref[...]
        o = jnp.maximum(jnp.dot(pooled, v1[...],
                                preferred_element_type=jnp.float32) + c1[...], 0.0)
        o = jnp.maximum(jnp.dot(o, v2[...],
                                preferred_element_type=jnp.float32) + c2[...], 0.0)
        out_ref[...] = jnp.dot(o, v3[...],
                               preferred_element_type=jnp.float32) + c3[...]


def kernel(flat, cu_seqlens, W1, b1, W2, b2, W3, b3, V1, c1, V2, c2, V3, c3):
    counts = (cu_seqlens[1:] - cu_seqlens[:-1]).astype(jnp.float32)
    inv = (1.0 / jnp.maximum(counts, 1.0)).reshape(_NSEG, 1)

    full = lambda a: pl.BlockSpec(a.shape, lambda i: (0,) * a.ndim)
    b1r, b2r, b3r = b1.reshape(1, -1), b2.reshape(1, -1), b3.reshape(1, -1)
    c1r, c2r, c3r = c1.reshape(1, -1), c2.reshape(1, -1), c3.reshape(1, -1)

    return pl.pallas_call(
        _fused,
        grid=(_GRID,),
        in_specs=[
            pl.BlockSpec(memory_space=pltpu.SMEM),
            pl.BlockSpec((_BS, _D), lambda i: (i, 0)),
            full(W1), full(b1r), full(W2), full(b2r), full(W3), full(b3r),
            full(V1), full(c1r), full(V2), full(c2r), full(V3), full(c3r),
            full(inv),
        ],
        out_specs=pl.BlockSpec((_NSEG, 25), lambda i: (0, 0)),
        out_shape=jax.ShapeDtypeStruct((_NSEG, 25), jnp.float32),
        scratch_shapes=[pltpu.VMEM((_NSEG, 128), jnp.float32)],
        compiler_params=pltpu.CompilerParams(
            dimension_semantics=("arbitrary",)),
    )(cu_seqlens, flat, W1, b1r, W2, b2r, W3, b3r,
      V1, c1r, V2, c2r, V3, c3r, inv)


# speedup vs baseline: 9.4974x; 1.0411x over previous
import jax
import jax.numpy as jnp
from jax.experimental import pallas as pl
from jax.experimental.pallas import tpu as pltpu

_TOT = 32768
_D = 3
_NSEG = 16
_BS = 8192
_GRID = _TOT // _BS


def _fused(cu_ref, flat_ref, w1, b1, w2, b2, w3, b3,
           v1, c1, v2, c2, v3, c3, inv_ref, out_ref, acc_ref):
    i = pl.program_id(0)

    @pl.when(i == 0)
    def _init():
        acc_ref[...] = jnp.zeros_like(acc_ref)

    x = flat_ref[...]
    h = jnp.maximum(jnp.dot(x, w1[...], preferred_element_type=jnp.float32)
                    + b1[...], 0.0)
    h = jnp.maximum(jnp.dot(h, w2[...], preferred_element_type=jnp.float32)
                    + b2[...], 0.0)
    h = jnp.maximum(jnp.dot(h, w3[...], preferred_element_type=jnp.float32)
                    + b3[...], 0.0)

    rows = i * _BS + jax.lax.broadcasted_iota(jnp.int32, (1, _BS), 1)
    seg = jnp.zeros((1, _BS), jnp.int32)
    for j in range(1, _NSEG):
        seg = seg + (rows >= cu_ref[j]).astype(jnp.int32)
    onehot_t = (seg == jax.lax.broadcasted_iota(jnp.int32, (_NSEG, _BS), 0)
                ).astype(jnp.float32)
    acc_ref[...] += jnp.dot(onehot_t, h, preferred_element_type=jnp.float32)

    @pl.when(i == _GRID - 1)
    def _finish():
        pooled = acc_ref[...] * inv_ref[...]
        o = jnp.maximum(jnp.dot(pooled, v1[...],
                                preferred_element_type=jnp.float32) + c1[...], 0.0)
        o = jnp.maximum(jnp.dot(o, v2[...],
                                preferred_element_type=jnp.float32) + c2[...], 0.0)
        out_ref[...] = jnp.dot(o, v3[...],
                               preferred_element_type=jnp.float32) + c3[...]


def kernel(flat, cu_seqlens, W1, b1, W2, b2, W3, b3, V1, c1, V2, c2, V3, c3):
    counts = (cu_seqlens[1:] - cu_seqlens[:-1]).astype(jnp.float32)
    inv = (1.0 / jnp.maximum(counts, 1.0)).reshape(_NSEG, 1)

    full = lambda a: pl.BlockSpec(a.shape, lambda i: (0,) * a.ndim)
    b1r, b2r, b3r = b1.reshape(1, -1), b2.reshape(1, -1), b3.reshape(1, -1)
    c1r, c2r, c3r = c1.reshape(1, -1), c2.reshape(1, -1), c3.reshape(1, -1)

    return pl.pallas_call(
        _fused,
        grid=(_GRID,),
        in_specs=[
            pl.BlockSpec(memory_space=pltpu.SMEM),
            pl.BlockSpec((_BS, _D), lambda i: (i, 0)),
            full(W1), full(b1r), full(W2), full(b2r), full(W3), full(b3r),
            full(V1), full(c1r), full(V2), full(c2r), full(V3), full(c3r),
            full(inv),
        ],
        out_specs=pl.BlockSpec((_NSEG, 25), lambda i: (0, 0)),
        out_shape=jax.ShapeDtypeStruct((_NSEG, 25), jnp.float32),
        scratch_shapes=[pltpu.VMEM((_NSEG, 128), jnp.float32)],
        compiler_params=pltpu.CompilerParams(
            dimension_semantics=("arbitrary",)),
    )(cu_seqlens, flat, W1, b1r, W2, b2r, W3, b3r,
      V1, c1r, V2, c2r, V3, c3r, inv)
